# Initial kernel scaffold; baseline (speedup 1.0000x reference)
import jax
import jax.numpy as jnp
from jax.experimental import pallas as pl


def kernel(mol_x, mol_edge_attr, prot_x, prot_evo, atom_type_emb, W_af1, W_af2, W_aa1, W_aa2, W_evo1, W_evo2, W_mol_pre, W_mol_post, W_prot_pre, W_prot_post, W_cq, W_ck, W_cv, W_mol_out1, W_mol_out2, W_prot_out1, W_prot_out2, W_mu1, W_mu2, W_sig1, W_sig2, atom_type, mol_edge_index, prot_edge_index, mol_batch, prot_batch):
    raise NotImplementedError("write your pallas kernel here")


# speedup vs baseline: 1.0016x; 1.0016x over previous
import functools
import jax
import jax.numpy as jnp
from jax.experimental import pallas as pl


H = 200
B = 256


def _layer_norm(x):
    mu = jnp.mean(x, axis=-1, keepdims=True)
    var = jnp.var(x, axis=-1, keepdims=True)
    return (x - mu) / jnp.sqrt(var + 1e-5)


def _mlp_norm(x, W1, W2):
    return _layer_norm(jax.nn.relu(x @ W1) @ W2)


def _segment_mean(x, seg, num):
    s = jax.ops.segment_sum(x, seg, num)
    c = jax.ops.segment_sum(jnp.ones((x.shape[0],), jnp.float32), seg, num)[:, None]
    return s / jnp.maximum(c, 1.0)


def _pna(x, ei, eattr, Wpre, Wpost):
    src, dst = ei[0], ei[1]
    parts = [x[src], x[dst]]
    if eattr is not None:
        parts.append(eattr)
    m = jax.nn.relu(jnp.concatenate(parts, axis=-1) @ Wpre)
    n = x.shape[0]
    ones = jnp.ones((ei.shape[1],), jnp.float32)
    cnt = jax.ops.segment_sum(ones, dst, n)[:, None]
    safe = jnp.maximum(cnt, 1.0)
    mean = jax.ops.segment_sum(m, dst, n) / safe
    mn = jax.ops.segment_min(m, dst, n)
    mx = jax.ops.segment_max(m, dst, n)
    has = cnt > 0
    mn = jnp.where(has, mn, 0.0)
    mx = jnp.where(has, mx, 0.0)
    sq = jax.ops.segment_sum(m * m, dst, n) / safe
    std = jnp.sqrt(jax.nn.relu(sq - mean * mean) + 1e-5)
    agg = jnp.concatenate([mean, mn, mx, std], axis=-1)
    log_deg = jnp.log(cnt + 1.0)
    amp = log_deg / jnp.maximum(jnp.mean(log_deg), 1e-6)
    lin = cnt / jnp.maximum(jnp.mean(cnt), 1.0)
    scaled = jnp.concatenate([agg, agg * amp, agg * lin], axis=-1)
    return _layer_norm(x + scaled @ Wpost)


def _head_kernel(z_ref, w1m_ref, w2m_ref, w1s_ref, w2s_ref, out_ref):
    z = z_ref[...]
    mu = jax.nn.relu(z @ w1m_ref[...]) @ w2m_ref[...]
    sig = jax.nn.relu(z @ w1s_ref[...]) @ w2s_ref[...]
    out_ref[...] = jnp.concatenate([mu, sig], axis=-1)


def _head(z, W_mu1, W_mu2, W_sig1, W_sig2):
    pad = jnp.zeros((H, 127), jnp.float32)
    w2m = jnp.concatenate([W_mu2, pad], axis=-1)
    w2s = jnp.concatenate([W_sig2, pad], axis=-1)
    out = pl.pallas_call(
        _head_kernel,
        out_shape=jax.ShapeDtypeStruct((B, 256), jnp.float32),
    )(z, W_mu1, w2m, W_sig1, w2s)
    return jnp.stack([out[:, 0], out[:, 128]], axis=-1)


def kernel(mol_x, mol_edge_attr, prot_x, prot_evo, atom_type_emb, W_af1, W_af2, W_aa1, W_aa2, W_evo1, W_evo2, W_mol_pre, W_mol_post, W_prot_pre, W_prot_post, W_cq, W_ck, W_cv, W_mol_out1, W_mol_out2, W_prot_out1, W_prot_out2, W_mu1, W_mu2, W_sig1, W_sig2, atom_type, mol_edge_index, prot_edge_index, mol_batch, prot_batch):
    h_mol = atom_type_emb[atom_type] + _mlp_norm(mol_x, W_af1, W_af2)
    h_prot = _mlp_norm(prot_x, W_aa1, W_aa2) + _mlp_norm(prot_evo, W_evo1, W_evo2)
    for _ in range(3):
        h_mol = _pna(h_mol, mol_edge_index, mol_edge_attr, W_mol_pre, W_mol_post)
        h_prot = _pna(h_prot, prot_edge_index, None, W_prot_pre, W_prot_post)
        r_pool = _segment_mean(h_prot, prot_batch, B)
        a_pool = _segment_mean(h_mol, mol_batch, B)
        q = h_mol @ W_cq
        k = (r_pool @ W_ck)[mol_batch]
        v = (r_pool @ W_cv)[mol_batch]
        gate = jax.nn.sigmoid(jnp.sum(q * k, axis=-1, keepdims=True) / jnp.sqrt(float(H)))
        h_mol = _layer_norm(h_mol + gate * v)
        q2 = h_prot @ W_cq
        k2 = (a_pool @ W_ck)[prot_batch]
        v2 = (a_pool @ W_cv)[prot_batch]
        gate2 = jax.nn.sigmoid(jnp.sum(q2 * k2, axis=-1, keepdims=True) / jnp.sqrt(float(H)))
        h_prot = _layer_norm(h_prot + gate2 * v2)
    mol_g = _mlp_norm(_segment_mean(h_mol, mol_batch, B), W_mol_out1, W_mol_out2)
    prot_g = _mlp_norm(_segment_mean(h_prot, prot_batch, B), W_prot_out1, W_prot_out2)
    z = jnp.concatenate([mol_g, prot_g], axis=-1)
    return _head(z, W_mu1, W_mu2, W_sig1, W_sig2)


# speedup vs baseline: 1.2087x; 1.2068x over previous
import functools
import jax
import jax.numpy as jnp
from jax import lax
from jax.experimental import pallas as pl
from jax.experimental.pallas import tpu as pltpu
from jax.experimental.pallas import tpu_sc as plsc

H = 200
B = 256
F = 256
G = 128
NWORK = 32
NEG = -3.0e38
POS = 3.0e38


def _pad_cols(x, f=F):
    return jnp.pad(x, ((0, 0), (0, f - x.shape[1])))


def _sc_aggregate(xs, xd, srcs, dsts, ests, perm, epre):
    n = xs.shape[0]
    has_e = epre is not None
    mesh = plsc.VectorSubcoreMesh(core_axis_name="c", subcore_axis_name="s")

    scratch = [
        pltpu.VMEM((48,), jnp.int32),
        pltpu.VMEM((G,), jnp.int32),
        pltpu.VMEM((G + 16,), jnp.int32),
        pltpu.VMEM((G, F), jnp.float32),
        pltpu.VMEM((G, F), jnp.float32),
        pltpu.VMEM((4, F), jnp.float32),
        pltpu.SMEM((8,), jnp.int32),
        pltpu.SemaphoreType.DMA,
    ]
    if has_e:
        scratch += [
            pltpu.VMEM((G,), jnp.int32),
            pltpu.VMEM((G, F), jnp.float32),
        ]

    def body(xs_h, xd_h, srcs_h, dsts_h, ests_h, *rest):
        if has_e:
            perm_h, epre_h, out_h, est_v, src_v, dst_v, xsr, xdr, acc, cur, sem, perm_v, eprer = rest
        else:
            out_h, est_v, src_v, dst_v, xsr, xdr, acc, cur, sem = rest
            perm_h = epre_h = perm_v = eprer = None

        wid = lax.axis_index("s") * 2 + lax.axis_index("c")
        pltpu.sync_copy(ests_h, est_v)
        ev = est_v[pl.ds(wid, 16)]
        e0 = ev[0]
        e1 = ev[1]
        eb0 = (e0 // 8) * 8
        nch = (e1 - eb0 + G - 1) // G
        cur[0] = -1

        def reset_acc():
            for s in range(F // 16):
                sl = pl.ds(s * 16, 16)
                acc[0, sl] = jnp.zeros((16,), jnp.float32)
                acc[1, sl] = jnp.full((16,), POS, jnp.float32)
                acc[2, sl] = jnp.full((16,), NEG, jnp.float32)
                acc[3, sl] = jnp.zeros((16,), jnp.float32)

        reset_acc()

        def chunk(c, carry):
            eb = pl.multiple_of(eb0 + c * G, 8)
            pltpu.sync_copy(srcs_h.at[pl.ds(eb, G)], src_v)
            pltpu.sync_copy(dsts_h.at[pl.ds(eb, G)], dst_v.at[pl.ds(0, G)])
            pltpu.async_copy(xs_h.at[src_v], xsr, sem).wait()
            pltpu.async_copy(xd_h.at[dst_v.at[pl.ds(0, G)]], xdr, sem).wait()
            if has_e:
                pltpu.sync_copy(perm_h.at[pl.ds(eb, G)], perm_v)
                pltpu.async_copy(epre_h.at[perm_v], eprer, sem).wait()

            def edge(e, carry2):
                i = e - eb
                d = dst_v[pl.ds(i, 16)][0]

                @pl.when(d != cur[0])
                def _():
                    @pl.when(cur[0] >= 0)
                    def _():
                        pltpu.sync_copy(acc, out_h.at[cur[0]])
                    reset_acc()
                    cur[0] = d

                for s in range(F // 16):
                    sl = pl.ds(s * 16, 16)
                    m = xsr[i, sl] + xdr[i, sl]
                    if has_e:
                        m = m + eprer[i, sl]
                    m = jnp.maximum(m, 0.0)
                    acc[0, sl] = acc[0, sl] + m
                    acc[1, sl] = jnp.minimum(acc[1, sl], m)
                    acc[2, sl] = jnp.maximum(acc[2, sl], m)
                    acc[3, sl] = acc[3, sl] + m * m

                return carry2

            lo = jnp.maximum(e0, eb)
            hi = jnp.minimum(e1, eb + G)
            lax.fori_loop(lo, hi, edge, 0)
            return carry

        lax.fori_loop(0, nch, chunk, 0)

        @pl.when(cur[0] >= 0)
        def _():
            pltpu.sync_copy(acc, out_h.at[cur[0]])

    kfn = functools.partial(
        pl.kernel,
        mesh=mesh,
        out_type=jax.ShapeDtypeStruct((n, 4, F), jnp.float32),
        scratch_types=scratch,
    )(body)
    if has_e:
        return kfn(xs, xd, srcs, dsts, ests, perm, epre)
    return kfn(xs, xd, srcs, dsts, ests)


def _prep_graph(ei, n):
    src, dst = ei[0].astype(jnp.int32), ei[1].astype(jnp.int32)
    e = src.shape[0]
    perm = jnp.argsort(dst)
    srcs = src[perm]
    dsts = dst[perm]
    rowptr = jnp.searchsorted(dsts, jnp.arange(n + 1, dtype=jnp.int32)).astype(jnp.int32)
    cnt = (rowptr[1:] - rowptr[:-1]).astype(jnp.float32)[:, None]
    nw = n // NWORK
    ests = rowptr[jnp.arange(NWORK + 1, dtype=jnp.int32) * nw]
    ests = jnp.pad(ests, (0, 48 - NWORK - 1))
    pad = G
    srcs = jnp.pad(srcs, (0, pad))
    dsts = jnp.pad(dsts, (0, pad))
    perm = jnp.pad(perm.astype(jnp.int32), (0, pad))
    return srcs, dsts, perm, ests, cnt


def _layer_norm(x):
    mu = jnp.mean(x, axis=-1, keepdims=True)
    var = jnp.var(x, axis=-1, keepdims=True)
    return (x - mu) / jnp.sqrt(var + 1e-5)


def _mlp_norm(x, W1, W2):
    return _layer_norm(jax.nn.relu(x @ W1) @ W2)


def _row_spec(r, c):
    return pl.BlockSpec((r, c), lambda i: (i, 0))


def _full_spec(shape):
    return pl.BlockSpec(shape, lambda i: tuple(0 for _ in shape))


def _mm_pallas(x, w, rblk=2048):
    n, kdim = x.shape
    cols = w.shape[1]

    def kern(x_ref, w_ref, o_ref):
        o_ref[...] = jnp.dot(x_ref[...], w_ref[...],
                             preferred_element_type=jnp.float32, precision=jax.lax.Precision.HIGHEST)

    return pl.pallas_call(
        kern,
        grid=(n // rblk,),
        in_specs=[_row_spec(rblk, kdim), _full_spec(w.shape)],
        out_specs=_row_spec(rblk, cols),
        out_shape=jax.ShapeDtypeStruct((n, cols), jnp.float32),
    )(x, w)


def _mlp_norm_pallas(x, w1, w2, add=None, rblk=1024):
    n, kdim = x.shape
    cols = w2.shape[1]
    have_add = add is not None

    def kern(*refs):
        if have_add:
            x_ref, w1_ref, w2_ref, a_ref, o_ref = refs
        else:
            x_ref, w1_ref, w2_ref, o_ref = refs
        h = jnp.dot(jax.nn.relu(jnp.dot(x_ref[...], w1_ref[...],
                                        preferred_element_type=jnp.float32, precision=jax.lax.Precision.HIGHEST)),
                    w2_ref[...], preferred_element_type=jnp.float32, precision=jax.lax.Precision.HIGHEST)
        h = _layer_norm(h)
        if have_add:
            h = h + a_ref[...]
        o_ref[...] = h

    specs = [_row_spec(rblk, kdim), _full_spec(w1.shape), _full_spec(w2.shape)]
    args = [x, w1, w2]
    if have_add:
        specs.append(_row_spec(rblk, add.shape[1]))
        args.append(add)
    return pl.pallas_call(
        kern,
        grid=(n // rblk,),
        in_specs=specs,
        out_specs=_row_spec(rblk, cols),
        out_shape=jax.ShapeDtypeStruct((n, cols), jnp.float32),
    )(*args)


def _init_mol_pallas(atom, emb, x, w1, w2, rblk=1024):
    n = x.shape[0]

    def kern(a_ref, e_ref, x_ref, w1_ref, w2_ref, o_ref):
        ids = a_ref[...]
        oh = (ids == jax.lax.broadcasted_iota(jnp.int32, (1, 20), 1)
              ).astype(jnp.float32)
        base = jnp.dot(oh, e_ref[...], preferred_element_type=jnp.float32, precision=jax.lax.Precision.HIGHEST)
        h = jnp.dot(jax.nn.relu(jnp.dot(x_ref[...], w1_ref[...],
                                        preferred_element_type=jnp.float32, precision=jax.lax.Precision.HIGHEST)),
                    w2_ref[...], preferred_element_type=jnp.float32, precision=jax.lax.Precision.HIGHEST)
        o_ref[...] = base + _layer_norm(h)

    return pl.pallas_call(
        kern,
        grid=(n // rblk,),
        in_specs=[_row_spec(rblk, 1), _full_spec(emb.shape),
                  _row_spec(rblk, x.shape[1]), _full_spec(w1.shape),
                  _full_spec(w2.shape)],
        out_specs=_row_spec(rblk, H),
        out_shape=jax.ShapeDtypeStruct((n, H), jnp.float32),
    )(atom[:, None].astype(jnp.int32), emb, x, w1, w2)


def _deg_pallas(cnt):
    n = cnt.shape[0]

    def kern(c_ref, amp_ref, lin_ref):
        c = c_ref[...]
        ld = jnp.log(c + 1.0)
        amp_ref[...] = ld / jnp.maximum(jnp.mean(ld), 1e-6)
        lin_ref[...] = c / jnp.maximum(jnp.mean(c), 1.0)

    return pl.pallas_call(
        kern,
        out_shape=[jax.ShapeDtypeStruct((n, 1), jnp.float32),
                   jax.ShapeDtypeStruct((n, 1), jnp.float32)],
    )(cnt)


def _post_pallas(agg4, cnt, amp, lin, h, wbig, rblk=1024):
    n = h.shape[0]

    def kern(a_ref, c_ref, am_ref, l_ref, h_ref, w_ref, o_ref):
        c = c_ref[...]
        has = c > 0.0
        rsafe = 1.0 / jnp.maximum(c, 1.0)
        s_ = a_ref[:, 0, :]
        mn = a_ref[:, 1, :]
        mx = a_ref[:, 2, :]
        sq = a_ref[:, 3, :]
        mean = jnp.where(has, s_ * rsafe, 0.0)
        mn = jnp.where(has, mn, 0.0)
        mx = jnp.where(has, mx, 0.0)
        msq = jnp.where(has, sq * rsafe, 0.0)
        std = jnp.sqrt(jax.nn.relu(msq - mean * mean) + 1e-5)
        agg = jnp.concatenate([mean, mn, mx, std], axis=-1)
        am = am_ref[...]
        l = l_ref[...]
        scaled = jnp.concatenate([agg, agg * am, agg * l], axis=-1)
        out = jnp.dot(scaled, w_ref[...], preferred_element_type=jnp.float32, precision=jax.lax.Precision.HIGHEST)
        o_ref[...] = _layer_norm(h_ref[...] + out)

    return pl.pallas_call(
        kern,
        grid=(n // rblk,),
        in_specs=[pl.BlockSpec((rblk, 4, F), lambda i: (i, 0, 0)),
                  _row_spec(rblk, 1), _row_spec(rblk, 1), _row_spec(rblk, 1),
                  _row_spec(rblk, H), _full_spec(wbig.shape)],
        out_specs=_row_spec(rblk, H),
        out_shape=jax.ShapeDtypeStruct((n, H), jnp.float32),
    )(agg4, cnt, amp, lin, h, wbig)


def _pool_pallas(h, batch, rblk=2048):
    n = h.shape[0]

    def kern(b_ref, h_ref, o_ref):
        pid = pl.program_id(0)

        @pl.when(pid == 0)
        def _():
            o_ref[...] = jnp.zeros_like(o_ref)

        ids = b_ref[...]
        oh = (ids == jax.lax.broadcasted_iota(jnp.int32, (1, B), 1)
              ).astype(jnp.float32)
        o_ref[...] += jax.lax.dot_general(
            oh, h_ref[...], (((0,), (0,)), ((), ())),
            preferred_element_type=jnp.float32, precision=jax.lax.Precision.HIGHEST)

    return pl.pallas_call(
        kern,
        grid=(n // rblk,),
        in_specs=[_row_spec(rblk, 1), _row_spec(rblk, H)],
        out_specs=_full_spec((B, H)),
        out_shape=jax.ShapeDtypeStruct((B, H), jnp.float32),
    )(batch[:, None], h)


def _attn_pallas(h, batch, pool_sum, pc, wq, wk, wv, rblk=2048):
    n = h.shape[0]

    def kern(h_ref, b_ref, p_ref, c_ref, wq_ref, wk_ref, wv_ref, o_ref):
        pool = p_ref[...] / jnp.maximum(c_ref[...], 1.0)
        k = jnp.dot(pool, wk_ref[...], preferred_element_type=jnp.float32, precision=jax.lax.Precision.HIGHEST)
        v = jnp.dot(pool, wv_ref[...], preferred_element_type=jnp.float32, precision=jax.lax.Precision.HIGHEST)
        ids = b_ref[...]
        oh = (ids == jax.lax.broadcasted_iota(jnp.int32, (1, B), 1)
              ).astype(jnp.float32)
        k_exp = jnp.dot(oh, k, preferred_element_type=jnp.float32, precision=jax.lax.Precision.HIGHEST)
        v_exp = jnp.dot(oh, v, preferred_element_type=jnp.float32, precision=jax.lax.Precision.HIGHEST)
        hh = h_ref[...]
        q = jnp.dot(hh, wq_ref[...], preferred_element_type=jnp.float32, precision=jax.lax.Precision.HIGHEST)
        gate = jax.nn.sigmoid(
            jnp.sum(q * k_exp, axis=-1, keepdims=True) / jnp.sqrt(float(H)))
        o_ref[...] = _layer_norm(hh + gate * v_exp)

    return pl.pallas_call(
        kern,
        grid=(n // rblk,),
        in_specs=[_row_spec(rblk, H), _row_spec(rblk, 1),
                  _full_spec((B, H)), _full_spec((B, 1)),
                  _full_spec(wq.shape), _full_spec(wk.shape),
                  _full_spec(wv.shape)],
        out_specs=_row_spec(rblk, H),
        out_shape=jax.ShapeDtypeStruct((n, H), jnp.float32),
    )(h, batch[:, None], pool_sum, pc, wq, wk, wv)


def _head_pallas(pm, pcm, pp, pcp, wmo1, wmo2, wpo1, wpo2,
                 wmu1a, wmu1b, wmu2, wsig1a, wsig1b, wsig2):
    def kern(pm_ref, pcm_ref, pp_ref, pcp_ref, wmo1_r, wmo2_r, wpo1_r,
             wpo2_r, wmu1a_r, wmu1b_r, wmu2_r, wsig1a_r, wsig1b_r,
             wsig2_r, o_ref):
        mg = pm_ref[...] / jnp.maximum(pcm_ref[...], 1.0)
        pg = pp_ref[...] / jnp.maximum(pcp_ref[...], 1.0)
        mol_g = _layer_norm(jnp.dot(jax.nn.relu(
            jnp.dot(mg, wmo1_r[...], preferred_element_type=jnp.float32, precision=jax.lax.Precision.HIGHEST)),
            wmo2_r[...], preferred_element_type=jnp.float32, precision=jax.lax.Precision.HIGHEST))
        prot_g = _layer_norm(jnp.dot(jax.nn.relu(
            jnp.dot(pg, wpo1_r[...], preferred_element_type=jnp.float32, precision=jax.lax.Precision.HIGHEST)),
            wpo2_r[...], preferred_element_type=jnp.float32, precision=jax.lax.Precision.HIGHEST))
        zmu = jax.nn.relu(
            jnp.dot(mol_g, wmu1a_r[...], preferred_element_type=jnp.float32, precision=jax.lax.Precision.HIGHEST)
            + jnp.dot(prot_g, wmu1b_r[...], preferred_element_type=jnp.float32, precision=jax.lax.Precision.HIGHEST))
        zsig = jax.nn.relu(
            jnp.dot(mol_g, wsig1a_r[...], preferred_element_type=jnp.float32, precision=jax.lax.Precision.HIGHEST)
            + jnp.dot(prot_g, wsig1b_r[...], preferred_element_type=jnp.float32, precision=jax.lax.Precision.HIGHEST))
        mu = jnp.dot(zmu, wmu2_r[...], preferred_element_type=jnp.float32, precision=jax.lax.Precision.HIGHEST)
        sig = jnp.dot(zsig, wsig2_r[...], preferred_element_type=jnp.float32, precision=jax.lax.Precision.HIGHEST)
        o_ref[...] = jnp.concatenate([mu, sig], axis=-1)

    return pl.pallas_call(
        kern,
        out_shape=jax.ShapeDtypeStruct((B, 256), jnp.float32),
    )(pm, pcm, pp, pcp, wmo1, wmo2, wpo1, wpo2,
      wmu1a, wmu1b, wmu2, wsig1a, wsig1b, wsig2)


def _pna_layer(h, prep, Wsd, Wbig, epre, cnt, amp, lin):
    srcs, dsts, perm, ests = prep
    xsxd = _mm_pallas(h, Wsd)
    xs = xsxd[:, :F]
    xd = xsxd[:, F:]
    agg4 = _sc_aggregate(xs, xd, srcs, dsts, ests,
                         perm if epre is not None else None, epre)
    return _post_pallas(agg4, cnt, amp, lin, h, Wbig)


def kernel(mol_x, mol_edge_attr, prot_x, prot_evo, atom_type_emb, W_af1, W_af2, W_aa1, W_aa2, W_evo1, W_evo2, W_mol_pre, W_mol_post, W_prot_pre, W_prot_post, W_cq, W_ck, W_cv, W_mol_out1, W_mol_out2, W_prot_out1, W_prot_out2, W_mu1, W_mu2, W_sig1, W_sig2, atom_type, mol_edge_index, prot_edge_index, mol_batch, prot_batch):
    n_mol = mol_x.shape[0]
    n_prot = prot_x.shape[0]

    m_srcs, m_dsts, m_perm, m_ests, m_cnt = _prep_graph(mol_edge_index, n_mol)
    p_srcs, p_dsts, p_perm, p_ests, p_cnt = _prep_graph(prot_edge_index, n_prot)
    mol_batch = mol_batch.astype(jnp.int32)
    prot_batch = prot_batch.astype(jnp.int32)
    pc_mol = (jnp.searchsorted(mol_batch, jnp.arange(B + 1, dtype=jnp.int32))[1:]
              - jnp.searchsorted(mol_batch, jnp.arange(B + 1, dtype=jnp.int32))[:-1]
              ).astype(jnp.float32)[:, None]
    pc_prot = (jnp.searchsorted(prot_batch, jnp.arange(B + 1, dtype=jnp.int32))[1:]
               - jnp.searchsorted(prot_batch, jnp.arange(B + 1, dtype=jnp.int32))[:-1]
               ).astype(jnp.float32)[:, None]

    m_amp, m_lin = _deg_pallas(m_cnt)
    p_amp, p_lin = _deg_pallas(p_cnt)

    def _sd_weights(wpre):
        ws = jnp.pad(wpre[:H], ((0, 0), (0, F - H)))
        wd = jnp.pad(wpre[H:2 * H], ((0, 0), (0, F - H)))
        return jnp.concatenate([ws, wd], axis=1)

    def _big_weights(wpost):
        wb = wpost.reshape(12, H, H)
        wb = jnp.pad(wb, ((0, 0), (0, F - H), (0, 0)))
        return wb.reshape(12 * F, H)

    Wsd_mol = _sd_weights(W_mol_pre)
    Wsd_prot = _sd_weights(W_prot_pre)
    Wbig_mol = _big_weights(W_mol_post)
    Wbig_prot = _big_weights(W_prot_post)
    Wm_e = jnp.pad(W_mol_pre[2 * H:], ((0, 0), (0, F - H)))
    epre_mol = _mm_pallas(mol_edge_attr, Wm_e)

    h_mol = _init_mol_pallas(atom_type, atom_type_emb, mol_x, W_af1, W_af2)
    h_aa = _mlp_norm_pallas(prot_x, W_aa1, W_aa2)
    h_prot = _mlp_norm_pallas(prot_evo, W_evo1, W_evo2, add=h_aa)

    m_prep = (m_srcs, m_dsts, m_perm, m_ests)
    p_prep = (p_srcs, p_dsts, p_perm, p_ests)

    for _ in range(3):
        h_mol = _pna_layer(h_mol, m_prep, Wsd_mol, Wbig_mol, epre_mol,
                           m_cnt, m_amp, m_lin)
        h_prot = _pna_layer(h_prot, p_prep, Wsd_prot, Wbig_prot, None,
                            p_cnt, p_amp, p_lin)
        ps_prot = _pool_pallas(h_prot, prot_batch)
        ps_mol = _pool_pallas(h_mol, mol_batch)
        h_mol = _attn_pallas(h_mol, mol_batch, ps_prot, pc_prot,
                             W_cq, W_ck, W_cv)
        h_prot = _attn_pallas(h_prot, prot_batch, ps_mol, pc_mol,
                              W_cq, W_ck, W_cv)

    pm = _pool_pallas(h_mol, mol_batch)
    pp = _pool_pallas(h_prot, prot_batch)
    out = _head_pallas(
        pm, pc_mol, pp, pc_prot, W_mol_out1, W_mol_out2,
        W_prot_out1, W_prot_out2,
        W_mu1[:H], W_mu1[H:], jnp.pad(W_mu2, ((0, 0), (0, 127))),
        W_sig1[:H], W_sig1[H:], jnp.pad(W_sig2, ((0, 0), (0, 127))))
    return jnp.stack([out[:, 0], out[:, 128]], axis=-1)


# speedup vs baseline: 1.2233x; 1.0120x over previous
import functools
import jax
import jax.numpy as jnp
from jax import lax
from jax.experimental import pallas as pl
from jax.experimental.pallas import tpu as pltpu
from jax.experimental.pallas import tpu_sc as plsc

H = 200
B = 256
F = 256
G = 128
NWORK = 32
NEG = -3.0e38
POS = 3.0e38


def _pad_cols(x, f=F):
    return jnp.pad(x, ((0, 0), (0, f - x.shape[1])))


def _sc_aggregate(xs, xd, srcs, dsts, ests, perm, epre):
    n = xs.shape[0]
    has_e = epre is not None
    mesh = plsc.VectorSubcoreMesh(core_axis_name="c", subcore_axis_name="s")

    scratch = [
        pltpu.VMEM((48,), jnp.int32),
        pltpu.VMEM((G,), jnp.int32),
        pltpu.VMEM((G + 16,), jnp.int32),
        pltpu.VMEM((G, F), jnp.float32),
        pltpu.VMEM((G, F), jnp.float32),
        pltpu.VMEM((4, F), jnp.float32),
        pltpu.SMEM((8,), jnp.int32),
        pltpu.SemaphoreType.DMA,
    ]
    if has_e:
        scratch += [
            pltpu.VMEM((G,), jnp.int32),
            pltpu.VMEM((G, F), jnp.float32),
        ]

    def body(xs_h, xd_h, srcs_h, dsts_h, ests_h, *rest):
        if has_e:
            perm_h, epre_h, out_h, est_v, src_v, dst_v, xsr, xdr, acc, cur, sem, perm_v, eprer = rest
        else:
            out_h, est_v, src_v, dst_v, xsr, xdr, acc, cur, sem = rest
            perm_h = epre_h = perm_v = eprer = None

        wid = lax.axis_index("s") * 2 + lax.axis_index("c")
        pltpu.sync_copy(ests_h, est_v)
        ev = est_v[pl.ds(wid, 16)]
        e0 = ev[0]
        e1 = ev[1]
        eb0 = (e0 // 8) * 8
        nch = (e1 - eb0 + G - 1) // G
        cur[0] = -1

        def reset_acc():
            for s in range(F // 16):
                sl = pl.ds(s * 16, 16)
                acc[0, sl] = jnp.zeros((16,), jnp.float32)
                acc[1, sl] = jnp.full((16,), POS, jnp.float32)
                acc[2, sl] = jnp.full((16,), NEG, jnp.float32)
                acc[3, sl] = jnp.zeros((16,), jnp.float32)

        reset_acc()

        def chunk(c, carry):
            eb = pl.multiple_of(eb0 + c * G, 8)
            pltpu.sync_copy(srcs_h.at[pl.ds(eb, G)], src_v)
            pltpu.sync_copy(dsts_h.at[pl.ds(eb, G)], dst_v.at[pl.ds(0, G)])
            if has_e:
                pltpu.sync_copy(perm_h.at[pl.ds(eb, G)], perm_v)
            c1 = pltpu.async_copy(xs_h.at[src_v], xsr, sem)
            c2 = pltpu.async_copy(xd_h.at[dst_v.at[pl.ds(0, G)]], xdr, sem)
            if has_e:
                c3 = pltpu.async_copy(epre_h.at[perm_v], eprer, sem)
            c1.wait()
            c2.wait()
            if has_e:
                c3.wait()

            def edge(e, carry2):
                i = e - eb
                d = dst_v[pl.ds(i, 16)][0]

                @pl.when(d != cur[0])
                def _():
                    @pl.when(cur[0] >= 0)
                    def _():
                        pltpu.sync_copy(acc, out_h.at[cur[0]])
                    reset_acc()
                    cur[0] = d

                for s in range(F // 16):
                    sl = pl.ds(s * 16, 16)
                    m = xsr[i, sl] + xdr[i, sl]
                    if has_e:
                        m = m + eprer[i, sl]
                    m = jnp.maximum(m, 0.0)
                    acc[0, sl] = acc[0, sl] + m
                    acc[1, sl] = jnp.minimum(acc[1, sl], m)
                    acc[2, sl] = jnp.maximum(acc[2, sl], m)
                    acc[3, sl] = acc[3, sl] + m * m

                return carry2

            lo = jnp.maximum(e0, eb)
            hi = jnp.minimum(e1, eb + G)
            lax.fori_loop(lo, hi, edge, 0)
            return carry

        lax.fori_loop(0, nch, chunk, 0)

        @pl.when(cur[0] >= 0)
        def _():
            pltpu.sync_copy(acc, out_h.at[cur[0]])

    kfn = functools.partial(
        pl.kernel,
        mesh=mesh,
        out_type=jax.ShapeDtypeStruct((n, 4, F), jnp.float32),
        scratch_types=scratch,
    )(body)
    if has_e:
        return kfn(xs, xd, srcs, dsts, ests, perm, epre)
    return kfn(xs, xd, srcs, dsts, ests)


def _prep_graph(ei, n):
    src, dst = ei[0].astype(jnp.int32), ei[1].astype(jnp.int32)
    e = src.shape[0]
    perm = jnp.argsort(dst)
    srcs = src[perm]
    dsts = dst[perm]
    rowptr = jnp.searchsorted(dsts, jnp.arange(n + 1, dtype=jnp.int32)).astype(jnp.int32)
    cnt = (rowptr[1:] - rowptr[:-1]).astype(jnp.float32)[:, None]
    nw = n // NWORK
    ests = rowptr[jnp.arange(NWORK + 1, dtype=jnp.int32) * nw]
    ests = jnp.pad(ests, (0, 48 - NWORK - 1))
    pad = G
    srcs = jnp.pad(srcs, (0, pad))
    dsts = jnp.pad(dsts, (0, pad))
    perm = jnp.pad(perm.astype(jnp.int32), (0, pad))
    return srcs, dsts, perm, ests, cnt


def _layer_norm(x):
    mu = jnp.mean(x, axis=-1, keepdims=True)
    var = jnp.var(x, axis=-1, keepdims=True)
    return (x - mu) / jnp.sqrt(var + 1e-5)


def _mlp_norm(x, W1, W2):
    return _layer_norm(jax.nn.relu(x @ W1) @ W2)


def _row_spec(r, c):
    return pl.BlockSpec((r, c), lambda i: (i, 0))


def _full_spec(shape):
    return pl.BlockSpec(shape, lambda i: tuple(0 for _ in shape))


def _mm_pallas(x, w, rblk=2048):
    n, kdim = x.shape
    cols = w.shape[1]

    def kern(x_ref, w_ref, o_ref):
        o_ref[...] = jnp.dot(x_ref[...], w_ref[...],
                             preferred_element_type=jnp.float32, precision=jax.lax.Precision.HIGHEST)

    return pl.pallas_call(
        kern,
        grid=(n // rblk,),
        in_specs=[_row_spec(rblk, kdim), _full_spec(w.shape)],
        out_specs=_row_spec(rblk, cols),
        out_shape=jax.ShapeDtypeStruct((n, cols), jnp.float32),
    )(x, w)


def _mlp_norm_pallas(x, w1, w2, add=None, rblk=1024):
    n, kdim = x.shape
    cols = w2.shape[1]
    have_add = add is not None

    def kern(*refs):
        if have_add:
            x_ref, w1_ref, w2_ref, a_ref, o_ref = refs
        else:
            x_ref, w1_ref, w2_ref, o_ref = refs
        h = jnp.dot(jax.nn.relu(jnp.dot(x_ref[...], w1_ref[...],
                                        preferred_element_type=jnp.float32, precision=jax.lax.Precision.HIGHEST)),
                    w2_ref[...], preferred_element_type=jnp.float32, precision=jax.lax.Precision.HIGHEST)
        h = _layer_norm(h)
        if have_add:
            h = h + a_ref[...]
        o_ref[...] = h

    specs = [_row_spec(rblk, kdim), _full_spec(w1.shape), _full_spec(w2.shape)]
    args = [x, w1, w2]
    if have_add:
        specs.append(_row_spec(rblk, add.shape[1]))
        args.append(add)
    return pl.pallas_call(
        kern,
        grid=(n // rblk,),
        in_specs=specs,
        out_specs=_row_spec(rblk, cols),
        out_shape=jax.ShapeDtypeStruct((n, cols), jnp.float32),
    )(*args)


def _init_mol_pallas(atom, emb, x, w1, w2, rblk=1024):
    n = x.shape[0]

    def kern(a_ref, e_ref, x_ref, w1_ref, w2_ref, o_ref):
        ids = a_ref[...]
        oh = (ids == jax.lax.broadcasted_iota(jnp.int32, (1, 20), 1)
              ).astype(jnp.float32)
        base = jnp.dot(oh, e_ref[...], preferred_element_type=jnp.float32, precision=jax.lax.Precision.HIGHEST)
        h = jnp.dot(jax.nn.relu(jnp.dot(x_ref[...], w1_ref[...],
                                        preferred_element_type=jnp.float32, precision=jax.lax.Precision.HIGHEST)),
                    w2_ref[...], preferred_element_type=jnp.float32, precision=jax.lax.Precision.HIGHEST)
        o_ref[...] = base + _layer_norm(h)

    return pl.pallas_call(
        kern,
        grid=(n // rblk,),
        in_specs=[_row_spec(rblk, 1), _full_spec(emb.shape),
                  _row_spec(rblk, x.shape[1]), _full_spec(w1.shape),
                  _full_spec(w2.shape)],
        out_specs=_row_spec(rblk, H),
        out_shape=jax.ShapeDtypeStruct((n, H), jnp.float32),
    )(atom[:, None].astype(jnp.int32), emb, x, w1, w2)


def _deg_pallas(cnt):
    n = cnt.shape[0]

    def kern(c_ref, amp_ref, lin_ref):
        c = c_ref[...]
        ld = jnp.log(c + 1.0)
        amp_ref[...] = ld / jnp.maximum(jnp.mean(ld), 1e-6)
        lin_ref[...] = c / jnp.maximum(jnp.mean(c), 1.0)

    return pl.pallas_call(
        kern,
        out_shape=[jax.ShapeDtypeStruct((n, 1), jnp.float32),
                   jax.ShapeDtypeStruct((n, 1), jnp.float32)],
    )(cnt)


def _post_pallas(agg4, cnt, amp, lin, h, wbig, rblk=1024):
    n = h.shape[0]

    def kern(a_ref, c_ref, am_ref, l_ref, h_ref, w_ref, o_ref):
        c = c_ref[...]
        has = c > 0.0
        rsafe = 1.0 / jnp.maximum(c, 1.0)
        s_ = a_ref[:, 0, :]
        mn = a_ref[:, 1, :]
        mx = a_ref[:, 2, :]
        sq = a_ref[:, 3, :]
        mean = jnp.where(has, s_ * rsafe, 0.0)
        mn = jnp.where(has, mn, 0.0)
        mx = jnp.where(has, mx, 0.0)
        msq = jnp.where(has, sq * rsafe, 0.0)
        std = jnp.sqrt(jax.nn.relu(msq - mean * mean) + 1e-5)
        agg = jnp.concatenate([mean, mn, mx, std], axis=-1)
        am = am_ref[...]
        l = l_ref[...]
        scaled = jnp.concatenate([agg, agg * am, agg * l], axis=-1)
        out = jnp.dot(scaled, w_ref[...], preferred_element_type=jnp.float32, precision=jax.lax.Precision.HIGHEST)
        o_ref[...] = _layer_norm(h_ref[...] + out)

    return pl.pallas_call(
        kern,
        grid=(n // rblk,),
        in_specs=[pl.BlockSpec((rblk, 4, F), lambda i: (i, 0, 0)),
                  _row_spec(rblk, 1), _row_spec(rblk, 1), _row_spec(rblk, 1),
                  _row_spec(rblk, H), _full_spec(wbig.shape)],
        out_specs=_row_spec(rblk, H),
        out_shape=jax.ShapeDtypeStruct((n, H), jnp.float32),
    )(agg4, cnt, amp, lin, h, wbig)


def _pool_pallas(h, batch, rblk=2048):
    n = h.shape[0]

    def kern(b_ref, h_ref, o_ref):
        pid = pl.program_id(0)

        @pl.when(pid == 0)
        def _():
            o_ref[...] = jnp.zeros_like(o_ref)

        ids = b_ref[...]
        oh = (ids == jax.lax.broadcasted_iota(jnp.int32, (1, B), 1)
              ).astype(jnp.float32)
        o_ref[...] += jax.lax.dot_general(
            oh, h_ref[...], (((0,), (0,)), ((), ())),
            preferred_element_type=jnp.float32,
            precision=jax.lax.Precision.HIGHEST)

    return pl.pallas_call(
        kern,
        grid=(n // rblk,),
        in_specs=[_row_spec(rblk, 1), _row_spec(rblk, H)],
        out_specs=_full_spec((B, H)),
        out_shape=jax.ShapeDtypeStruct((B, H), jnp.float32),
    )(batch[:, None], h)


def _attn_pallas(h, batch, pool_sum, pc, wq, wk, wv, rblk=2048):
    n = h.shape[0]

    def kern(h_ref, b_ref, p_ref, c_ref, wq_ref, wk_ref, wv_ref, o_ref):
        pool = p_ref[...] / jnp.maximum(c_ref[...], 1.0)
        k = jnp.dot(pool, wk_ref[...], preferred_element_type=jnp.float32, precision=jax.lax.Precision.HIGHEST)
        v = jnp.dot(pool, wv_ref[...], preferred_element_type=jnp.float32, precision=jax.lax.Precision.HIGHEST)
        ids = b_ref[...]
        oh = (ids == jax.lax.broadcasted_iota(jnp.int32, (1, B), 1)
              ).astype(jnp.float32)
        k_exp = jnp.dot(oh, k, preferred_element_type=jnp.float32, precision=jax.lax.Precision.HIGHEST)
        v_exp = jnp.dot(oh, v, preferred_element_type=jnp.float32, precision=jax.lax.Precision.HIGHEST)
        hh = h_ref[...]
        q = jnp.dot(hh, wq_ref[...], preferred_element_type=jnp.float32, precision=jax.lax.Precision.HIGHEST)
        gate = jax.nn.sigmoid(
            jnp.sum(q * k_exp, axis=-1, keepdims=True) / jnp.sqrt(float(H)))
        o_ref[...] = _layer_norm(hh + gate * v_exp)

    return pl.pallas_call(
        kern,
        grid=(n // rblk,),
        in_specs=[_row_spec(rblk, H), _row_spec(rblk, 1),
                  _full_spec((B, H)), _full_spec((B, 1)),
                  _full_spec(wq.shape), _full_spec(wk.shape),
                  _full_spec(wv.shape)],
        out_specs=_row_spec(rblk, H),
        out_shape=jax.ShapeDtypeStruct((n, H), jnp.float32),
    )(h, batch[:, None], pool_sum, pc, wq, wk, wv)


def _head_pallas(pm, pcm, pp, pcp, wmo1, wmo2, wpo1, wpo2,
                 wmu1a, wmu1b, wmu2, wsig1a, wsig1b, wsig2):
    def kern(pm_ref, pcm_ref, pp_ref, pcp_ref, wmo1_r, wmo2_r, wpo1_r,
             wpo2_r, wmu1a_r, wmu1b_r, wmu2_r, wsig1a_r, wsig1b_r,
             wsig2_r, o_ref):
        mg = pm_ref[...] / jnp.maximum(pcm_ref[...], 1.0)
        pg = pp_ref[...] / jnp.maximum(pcp_ref[...], 1.0)
        mol_g = _layer_norm(jnp.dot(jax.nn.relu(
            jnp.dot(mg, wmo1_r[...], preferred_element_type=jnp.float32, precision=jax.lax.Precision.HIGHEST)),
            wmo2_r[...], preferred_element_type=jnp.float32, precision=jax.lax.Precision.HIGHEST))
        prot_g = _layer_norm(jnp.dot(jax.nn.relu(
            jnp.dot(pg, wpo1_r[...], preferred_element_type=jnp.float32, precision=jax.lax.Precision.HIGHEST)),
            wpo2_r[...], preferred_element_type=jnp.float32, precision=jax.lax.Precision.HIGHEST))
        zmu = jax.nn.relu(
            jnp.dot(mol_g, wmu1a_r[...], preferred_element_type=jnp.float32, precision=jax.lax.Precision.HIGHEST)
            + jnp.dot(prot_g, wmu1b_r[...], preferred_element_type=jnp.float32, precision=jax.lax.Precision.HIGHEST))
        zsig = jax.nn.relu(
            jnp.dot(mol_g, wsig1a_r[...], preferred_element_type=jnp.float32, precision=jax.lax.Precision.HIGHEST)
            + jnp.dot(prot_g, wsig1b_r[...], preferred_element_type=jnp.float32, precision=jax.lax.Precision.HIGHEST))
        mu = jnp.dot(zmu, wmu2_r[...], preferred_element_type=jnp.float32, precision=jax.lax.Precision.HIGHEST)
        sig = jnp.dot(zsig, wsig2_r[...], preferred_element_type=jnp.float32, precision=jax.lax.Precision.HIGHEST)
        o_ref[...] = jnp.concatenate([mu, sig], axis=-1)

    return pl.pallas_call(
        kern,
        out_shape=jax.ShapeDtypeStruct((B, 256), jnp.float32),
    )(pm, pcm, pp, pcp, wmo1, wmo2, wpo1, wpo2,
      wmu1a, wmu1b, wmu2, wsig1a, wsig1b, wsig2)


def _pna_layer(h, prep, Wsd, Wbig, epre, cnt, amp, lin):
    srcs, dsts, perm, ests = prep
    xsxd = _mm_pallas(h, Wsd)
    xs = xsxd[:, :F]
    xd = xsxd[:, F:]
    agg4 = _sc_aggregate(xs, xd, srcs, dsts, ests,
                         perm if epre is not None else None, epre)
    return _post_pallas(agg4, cnt, amp, lin, h, Wbig)


def kernel(mol_x, mol_edge_attr, prot_x, prot_evo, atom_type_emb, W_af1, W_af2, W_aa1, W_aa2, W_evo1, W_evo2, W_mol_pre, W_mol_post, W_prot_pre, W_prot_post, W_cq, W_ck, W_cv, W_mol_out1, W_mol_out2, W_prot_out1, W_prot_out2, W_mu1, W_mu2, W_sig1, W_sig2, atom_type, mol_edge_index, prot_edge_index, mol_batch, prot_batch):
    n_mol = mol_x.shape[0]
    n_prot = prot_x.shape[0]

    m_srcs, m_dsts, m_perm, m_ests, m_cnt = _prep_graph(mol_edge_index, n_mol)
    p_srcs, p_dsts, p_perm, p_ests, p_cnt = _prep_graph(prot_edge_index, n_prot)
    mol_batch = mol_batch.astype(jnp.int32)
    prot_batch = prot_batch.astype(jnp.int32)
    pc_mol = (jnp.searchsorted(mol_batch, jnp.arange(B + 1, dtype=jnp.int32))[1:]
              - jnp.searchsorted(mol_batch, jnp.arange(B + 1, dtype=jnp.int32))[:-1]
              ).astype(jnp.float32)[:, None]
    pc_prot = (jnp.searchsorted(prot_batch, jnp.arange(B + 1, dtype=jnp.int32))[1:]
               - jnp.searchsorted(prot_batch, jnp.arange(B + 1, dtype=jnp.int32))[:-1]
               ).astype(jnp.float32)[:, None]

    m_amp, m_lin = _deg_pallas(m_cnt)
    p_amp, p_lin = _deg_pallas(p_cnt)

    def _sd_weights(wpre):
        ws = jnp.pad(wpre[:H], ((0, 0), (0, F - H)))
        wd = jnp.pad(wpre[H:2 * H], ((0, 0), (0, F - H)))
        return jnp.concatenate([ws, wd], axis=1)

    def _big_weights(wpost):
        wb = wpost.reshape(12, H, H)
        wb = jnp.pad(wb, ((0, 0), (0, F - H), (0, 0)))
        return wb.reshape(12 * F, H)

    Wsd_mol = _sd_weights(W_mol_pre)
    Wsd_prot = _sd_weights(W_prot_pre)
    Wbig_mol = _big_weights(W_mol_post)
    Wbig_prot = _big_weights(W_prot_post)
    Wm_e = jnp.pad(W_mol_pre[2 * H:], ((0, 0), (0, F - H)))
    epre_mol = _mm_pallas(mol_edge_attr, Wm_e)

    h_mol = _init_mol_pallas(atom_type, atom_type_emb, mol_x, W_af1, W_af2)
    h_aa = _mlp_norm_pallas(prot_x, W_aa1, W_aa2)
    h_prot = _mlp_norm_pallas(prot_evo, W_evo1, W_evo2, add=h_aa)

    m_prep = (m_srcs, m_dsts, m_perm, m_ests)
    p_prep = (p_srcs, p_dsts, p_perm, p_ests)

    for _ in range(3):
        h_mol = _pna_layer(h_mol, m_prep, Wsd_mol, Wbig_mol, epre_mol,
                           m_cnt, m_amp, m_lin)
        h_prot = _pna_layer(h_prot, p_prep, Wsd_prot, Wbig_prot, None,
                            p_cnt, p_amp, p_lin)
        ps_prot = _pool_pallas(h_prot, prot_batch)
        ps_mol = _pool_pallas(h_mol, mol_batch)
        h_mol = _attn_pallas(h_mol, mol_batch, ps_prot, pc_prot,
                             W_cq, W_ck, W_cv)
        h_prot = _attn_pallas(h_prot, prot_batch, ps_mol, pc_mol,
                              W_cq, W_ck, W_cv)

    pm = _pool_pallas(h_mol, mol_batch)
    pp = _pool_pallas(h_prot, prot_batch)
    out = _head_pallas(
        pm, pc_mol, pp, pc_prot, W_mol_out1, W_mol_out2,
        W_prot_out1, W_prot_out2,
        W_mu1[:H], W_mu1[H:], jnp.pad(W_mu2, ((0, 0), (0, 127))),
        W_sig1[:H], W_sig1[H:], jnp.pad(W_sig2, ((0, 0), (0, 127))))
    return jnp.stack([out[:, 0], out[:, 128]], axis=-1)
